# baseline (speedup 1.0000x reference)
import functools

import jax
import jax.numpy as jnp
from jax import lax
from jax.experimental import pallas as pl
from jax.experimental.pallas import tpu as pltpu
from jax.experimental.pallas import tpu_sc as plsc

_B = 16384
_F = 128
_L = 16
_C = 128
_NBUF = 2


def _dots_body(u_idx, i_idx, u_emb, i_emb, u_bias, i_bias, out,
               uidx_v, iidx_v, urows_v, irows_v, dots_v, ub2_v, ib2_v,
               sems, sem_b, *, rows_per_worker, num_cores):
    wid = lax.axis_index("s") * num_cores + lax.axis_index("c")
    nchunks = rows_per_worker // _C

    def start_chunk(c, b):
        base = wid * rows_per_worker + c * _C
        pltpu.sync_copy(u_idx.at[pl.ds(base, _C)], uidx_v.at[b])
        pltpu.sync_copy(i_idx.at[pl.ds(base, _C)], iidx_v.at[b])
        return [
            pltpu.async_copy(u_emb.at[uidx_v.at[b]], urows_v.at[b], sems.at[b]),
            pltpu.async_copy(i_emb.at[iidx_v.at[b]], irows_v.at[b], sems.at[b]),
        ]

    cps = {}
    for b in range(_NBUF):
        cps[b] = start_chunk(b, b)

    lane_iota = lax.iota(jnp.int32, _L)
    zero16 = jnp.zeros((_L,), jnp.int32)

    for c in range(nchunks):
        b = c % _NBUF
        for cp in cps[b]:
            cp.wait()

        def bias_fetch(g, _):
            gbase = g * _L
            uvec = uidx_v[b, pl.ds(gbase, _L)]
            ivec = iidx_v[b, pl.ds(gbase, _L)]
            for r in range(_L):
                pltpu.make_async_copy(u_bias.at[pl.ds(uvec[r], 1), :],
                                      ub2_v.at[pl.ds(gbase + r, 1), :],
                                      sem_b).start()
                pltpu.make_async_copy(i_bias.at[pl.ds(ivec[r], 1), :],
                                      ib2_v.at[pl.ds(gbase + r, 1), :],
                                      sem_b).start()
            return 0

        lax.fori_loop(0, _C // _L, bias_fetch, 0)

        def zero_body(g, _):
            dots_v[pl.ds(g * _L, _L)] = jnp.zeros((_L,), jnp.float32)
            return 0

        lax.fori_loop(0, _C // _L, zero_body, 0)

        uv = urows_v.at[b]
        iv = irows_v.at[b]

        def row_group_body(g, _):
            gbase = g * _L
            for r in range(_L):
                row = gbase + r
                acc = uv[row, pl.ds(0, _L)] * iv[row, pl.ds(0, _L)]
                for k in range(1, _F // _L):
                    acc += (uv[row, pl.ds(k * _L, _L)]
                            * iv[row, pl.ds(k * _L, _L)])
                plsc.addupdate_scatter(dots_v,
                                       [jnp.full((_L,), row, jnp.int32)], acc)
            return 0

        lax.fori_loop(0, _C // _L, row_group_body, 0)

        pltpu.make_async_copy(u_bias.at[pl.ds(0, _C), :], ub2_v, sem_b).wait()
        pltpu.make_async_copy(i_bias.at[pl.ds(0, _C), :], ib2_v, sem_b).wait()

        def sig_body(g, _):
            gslice = pl.ds(g * _L, _L)
            gidx = jnp.full((_L,), g * _L, jnp.int32) + lane_iota
            ub = plsc.load_gather(ub2_v, [gidx, zero16])
            ib = plsc.load_gather(ib2_v, [gidx, zero16])
            pred = dots_v[gslice] + ub + ib
            dots_v[gslice] = 1.0 / (1.0 + jnp.exp(-pred))
            return 0

        lax.fori_loop(0, _C // _L, sig_body, 0)
        base = wid * rows_per_worker + c * _C
        pltpu.sync_copy(dots_v, out.at[pl.ds(base, _C)])
        if c + _NBUF < nchunks:
            cps[b] = start_chunk(c + _NBUF, b)


@functools.cache
def _build():
    info = plsc.get_sparse_core_info()
    num_workers = info.num_cores * info.num_subcores
    rpw = _B // num_workers
    mesh = plsc.VectorSubcoreMesh(core_axis_name="c", subcore_axis_name="s")
    params = pltpu.CompilerParams(needs_layout_passes=False)

    return pl.kernel(
        functools.partial(_dots_body, rows_per_worker=rpw,
                          num_cores=info.num_cores),
        out_type=jax.ShapeDtypeStruct((_B,), jnp.float32),
        mesh=mesh,
        compiler_params=params,
        scratch_types=[
            pltpu.VMEM((_NBUF, _C), jnp.int32),
            pltpu.VMEM((_NBUF, _C), jnp.int32),
            pltpu.VMEM((_NBUF, _C, _F), jnp.float32),
            pltpu.VMEM((_NBUF, _C, _F), jnp.float32),
            pltpu.VMEM((_C,), jnp.float32),
            pltpu.VMEM((_C, 1), jnp.float32),
            pltpu.VMEM((_C, 1), jnp.float32),
            pltpu.SemaphoreType.DMA((_NBUF,)),
            pltpu.SemaphoreType.DMA,
        ],
    )


def kernel(u_idx, i_idx, u_emb, i_emb, u_bias, i_bias):
    return _build()(u_idx.astype(jnp.int32), i_idx.astype(jnp.int32),
                    u_emb, i_emb, u_bias, i_bias)


# speedup vs baseline: 4.1466x; 4.1466x over previous
import functools

import jax
import jax.numpy as jnp
from jax import lax
from jax.experimental import pallas as pl
from jax.experimental.pallas import tpu as pltpu
from jax.experimental.pallas import tpu_sc as plsc

_B = 16384
_F = 128
_L = 16
_C = 128
_NBUF = 2


def _dots_body(u_idx, i_idx, u_emb, i_emb, dots,
               uidx_v, iidx_v, urows_v, irows_v, dots_v, sems,
               *, rows_per_worker, num_cores):
    wid = lax.axis_index("s") * num_cores + lax.axis_index("c")
    nchunks = rows_per_worker // _C

    def start_chunk(c, b):
        base = wid * rows_per_worker + c * _C
        pltpu.sync_copy(u_idx.at[pl.ds(base, _C)], uidx_v.at[b])
        pltpu.sync_copy(i_idx.at[pl.ds(base, _C)], iidx_v.at[b])
        return [
            pltpu.async_copy(u_emb.at[uidx_v.at[b]], urows_v.at[b], sems.at[b]),
            pltpu.async_copy(i_emb.at[iidx_v.at[b]], irows_v.at[b], sems.at[b]),
        ]

    cps = {}
    for b in range(_NBUF):
        cps[b] = start_chunk(b, b)

    for c in range(nchunks):
        b = c % _NBUF
        for cp in cps[b]:
            cp.wait()

        def zero_body(g, _):
            dots_v[pl.ds(g * _L, _L)] = jnp.zeros((_L,), jnp.float32)
            return 0

        lax.fori_loop(0, _C // _L, zero_body, 0)

        uv = urows_v.at[b]
        iv = irows_v.at[b]

        def row_group_body(g, _):
            gbase = g * _L
            for r in range(_L):
                row = gbase + r
                acc = uv[row, pl.ds(0, _L)] * iv[row, pl.ds(0, _L)]
                for k in range(1, _F // _L):
                    acc += (uv[row, pl.ds(k * _L, _L)]
                            * iv[row, pl.ds(k * _L, _L)])
                plsc.addupdate_scatter(dots_v,
                                       [jnp.full((_L,), row, jnp.int32)], acc)
            return 0

        lax.fori_loop(0, _C // _L, row_group_body, 0)
        base = wid * rows_per_worker + c * _C
        pltpu.sync_copy(dots_v, dots.at[pl.ds(base, _C)])
        if c + _NBUF < nchunks:
            cps[b] = start_chunk(c + _NBUF, b)


def _bias_body(u_idx, i_idx, dots, u_bias, i_bias, out,
               uidx_v, iidx_v, ub_v, ib_v, dots_v, sem_i, sem_d,
               *, rows_per_worker, num_cores):
    wid = lax.axis_index("s") * num_cores + lax.axis_index("c")
    nchunks = rows_per_worker // _C

    def stage(c, b):
        base = wid * rows_per_worker + c * _C
        return [
            pltpu.async_copy(u_idx.at[pl.ds(base, _C)], uidx_v.at[b],
                             sem_i.at[b]),
            pltpu.async_copy(i_idx.at[pl.ds(base, _C)], iidx_v.at[b],
                             sem_i.at[b]),
        ], pltpu.async_copy(dots.at[pl.ds(base, _C)], dots_v.at[b],
                            sem_d.at[b])

    staged = {}
    for b in range(_NBUF):
        staged[b] = stage(b, b)

    for c in range(nchunks):
        b = c % _NBUF
        idx_cps, dots_cp = staged[b]
        for cp in idx_cps:
            cp.wait()
        gath = [
            pltpu.async_copy(u_bias.at[uidx_v.at[b]], ub_v.at[b], sem_d.at[b]),
            pltpu.async_copy(i_bias.at[iidx_v.at[b]], ib_v.at[b], sem_d.at[b]),
        ]
        dots_cp.wait()
        for cp in gath:
            cp.wait()

        dv = dots_v.at[b]
        ubv = ub_v.at[b]
        ibv = ib_v.at[b]

        def group_body(g, _):
            gslice = pl.ds(g * _L, _L)
            pred = dv[gslice] + ubv[gslice] + ibv[gslice]
            dv[gslice] = 1.0 / (1.0 + jnp.exp(-pred))
            return 0

        lax.fori_loop(0, _C // _L, group_body, 0)
        base = wid * rows_per_worker + c * _C
        pltpu.sync_copy(dots_v.at[b], out.at[pl.ds(base, _C)])
        if c + _NBUF < nchunks:
            staged[b] = stage(c + _NBUF, b)


@functools.cache
def _build():
    info = plsc.get_sparse_core_info()
    num_workers = info.num_cores * info.num_subcores
    rpw = _B // num_workers
    mesh = plsc.VectorSubcoreMesh(core_axis_name="c", subcore_axis_name="s")
    params = pltpu.CompilerParams(needs_layout_passes=False)

    dots_k = pl.kernel(
        functools.partial(_dots_body, rows_per_worker=rpw,
                          num_cores=info.num_cores),
        out_type=jax.ShapeDtypeStruct((_B,), jnp.float32),
        mesh=mesh,
        compiler_params=params,
        scratch_types=[
            pltpu.VMEM((_NBUF, _C), jnp.int32),
            pltpu.VMEM((_NBUF, _C), jnp.int32),
            pltpu.VMEM((_NBUF, _C, _F), jnp.float32),
            pltpu.VMEM((_NBUF, _C, _F), jnp.float32),
            pltpu.VMEM((_C,), jnp.float32),
            pltpu.SemaphoreType.DMA((_NBUF,)),
        ],
    )
    bias_k = pl.kernel(
        functools.partial(_bias_body, rows_per_worker=rpw,
                          num_cores=info.num_cores),
        out_type=jax.ShapeDtypeStruct((_B,), jnp.float32),
        mesh=mesh,
        compiler_params=params,
        scratch_types=[
            pltpu.VMEM((_NBUF, _C), jnp.int32),
            pltpu.VMEM((_NBUF, _C), jnp.int32),
            pltpu.VMEM((_NBUF, _C), jnp.float32),
            pltpu.VMEM((_NBUF, _C), jnp.float32),
            pltpu.VMEM((_NBUF, _C), jnp.float32),
            pltpu.SemaphoreType.DMA((_NBUF,)),
            pltpu.SemaphoreType.DMA((_NBUF,)),
        ],
    )
    return dots_k, bias_k


def kernel(u_idx, i_idx, u_emb, i_emb, u_bias, i_bias):
    dots_k, bias_k = _build()
    ui = u_idx.astype(jnp.int32)
    ii = i_idx.astype(jnp.int32)
    dots = dots_k(ui, ii, u_emb, i_emb)
    return bias_k(ui, ii, dots, u_bias.reshape(-1), i_bias.reshape(-1))


# speedup vs baseline: 4.1488x; 1.0006x over previous
import functools

import jax
import jax.numpy as jnp
from jax import lax
from jax.experimental import pallas as pl
from jax.experimental.pallas import tpu as pltpu
from jax.experimental.pallas import tpu_sc as plsc

_B = 16384
_F = 128
_L = 16
_C = 128
_NBUF = 2


def _dots_body(u_idx, i_idx, u_emb, i_emb, dots,
               uidx_v, iidx_v, urows_v, irows_v, dots_v, sems,
               *, rows_per_worker, num_cores):
    wid = lax.axis_index("s") * num_cores + lax.axis_index("c")
    nchunks = rows_per_worker // _C

    def start_chunk(c, b):
        base = wid * rows_per_worker + c * _C
        pltpu.sync_copy(u_idx.at[pl.ds(base, _C)], uidx_v.at[b])
        pltpu.sync_copy(i_idx.at[pl.ds(base, _C)], iidx_v.at[b])
        return [
            pltpu.async_copy(u_emb.at[uidx_v.at[b]], urows_v.at[b], sems.at[b]),
            pltpu.async_copy(i_emb.at[iidx_v.at[b]], irows_v.at[b], sems.at[b]),
        ]

    cps = {}
    for b in range(_NBUF):
        cps[b] = start_chunk(b, b)

    for c in range(nchunks):
        b = c % _NBUF
        for cp in cps[b]:
            cp.wait()

        def zero_body(g, _):
            dots_v[pl.ds(g * _L, _L)] = jnp.zeros((_L,), jnp.float32)
            return 0

        lax.fori_loop(0, _C // _L, zero_body, 0)

        uv = urows_v.at[b]
        iv = irows_v.at[b]

        def row_group_body(g, _):
            gbase = g * _L
            for r in range(_L):
                row = gbase + r
                acc = uv[row, pl.ds(0, _L)] * iv[row, pl.ds(0, _L)]
                for k in range(1, _F // _L):
                    acc += (uv[row, pl.ds(k * _L, _L)]
                            * iv[row, pl.ds(k * _L, _L)])
                plsc.addupdate_scatter(dots_v,
                                       [jnp.full((_L,), row, jnp.int32)], acc)
            return 0

        lax.fori_loop(0, _C // _L, row_group_body, 0)
        base = wid * rows_per_worker + c * _C
        pltpu.sync_copy(dots_v, dots.at[pl.ds(base, _C)])
        if c + _NBUF < nchunks:
            cps[b] = start_chunk(c + _NBUF, b)


def _bias_body(u_idx, i_idx, dots, u_bias, i_bias, out,
               uidx_v, iidx_v, ub_v, ib_v, dots_v, sem_i, sem_d,
               *, rows_per_worker, num_cores):
    wid = lax.axis_index("s") * num_cores + lax.axis_index("c")
    nchunks = rows_per_worker // _C
    nb = nchunks

    def stage(c, b):
        base = wid * rows_per_worker + c * _C
        return [
            pltpu.async_copy(u_idx.at[pl.ds(base, _C)], uidx_v.at[b],
                             sem_i.at[b]),
            pltpu.async_copy(i_idx.at[pl.ds(base, _C)], iidx_v.at[b],
                             sem_i.at[b]),
        ], pltpu.async_copy(dots.at[pl.ds(base, _C)], dots_v.at[b],
                            sem_d.at[b])

    staged = {}
    for b in range(nb):
        staged[b] = stage(b, b)

    for c in range(nchunks):
        b = c % nb
        idx_cps, dots_cp = staged[b]
        for cp in idx_cps:
            cp.wait()
        gath = [
            pltpu.async_copy(u_bias.at[uidx_v.at[b]], ub_v.at[b], sem_d.at[b]),
            pltpu.async_copy(i_bias.at[iidx_v.at[b]], ib_v.at[b], sem_d.at[b]),
        ]
        dots_cp.wait()
        for cp in gath:
            cp.wait()

        dv = dots_v.at[b]
        ubv = ub_v.at[b]
        ibv = ib_v.at[b]

        def group_body(g, _):
            gslice = pl.ds(g * _L, _L)
            pred = dv[gslice] + ubv[gslice] + ibv[gslice]
            dv[gslice] = 1.0 / (1.0 + jnp.exp(-pred))
            return 0

        lax.fori_loop(0, _C // _L, group_body, 0)
        base = wid * rows_per_worker + c * _C
        pltpu.sync_copy(dots_v.at[b], out.at[pl.ds(base, _C)])
        if c + nb < nchunks:
            staged[b] = stage(c + nb, b)


@functools.cache
def _build():
    info = plsc.get_sparse_core_info()
    num_workers = info.num_cores * info.num_subcores
    rpw = _B // num_workers
    mesh = plsc.VectorSubcoreMesh(core_axis_name="c", subcore_axis_name="s")
    params = pltpu.CompilerParams(needs_layout_passes=False)

    dots_k = pl.kernel(
        functools.partial(_dots_body, rows_per_worker=rpw,
                          num_cores=info.num_cores),
        out_type=jax.ShapeDtypeStruct((_B,), jnp.float32),
        mesh=mesh,
        compiler_params=params,
        scratch_types=[
            pltpu.VMEM((_NBUF, _C), jnp.int32),
            pltpu.VMEM((_NBUF, _C), jnp.int32),
            pltpu.VMEM((_NBUF, _C, _F), jnp.float32),
            pltpu.VMEM((_NBUF, _C, _F), jnp.float32),
            pltpu.VMEM((_C,), jnp.float32),
            pltpu.SemaphoreType.DMA((_NBUF,)),
        ],
    )
    bias_k = pl.kernel(
        functools.partial(_bias_body, rows_per_worker=rpw,
                          num_cores=info.num_cores),
        out_type=jax.ShapeDtypeStruct((_B,), jnp.float32),
        mesh=mesh,
        compiler_params=params,
        scratch_types=[
            pltpu.VMEM((4, _C), jnp.int32),
            pltpu.VMEM((4, _C), jnp.int32),
            pltpu.VMEM((4, _C), jnp.float32),
            pltpu.VMEM((4, _C), jnp.float32),
            pltpu.VMEM((4, _C), jnp.float32),
            pltpu.SemaphoreType.DMA((4,)),
            pltpu.SemaphoreType.DMA((4,)),
        ],
    )
    return dots_k, bias_k


def kernel(u_idx, i_idx, u_emb, i_emb, u_bias, i_bias):
    dots_k, bias_k = _build()
    ui = u_idx.astype(jnp.int32)
    ii = i_idx.astype(jnp.int32)
    dots = dots_k(ui, ii, u_emb, i_emb)
    return bias_k(ui, ii, dots, u_bias.reshape(-1), i_bias.reshape(-1))
